# baseline (speedup 1.0000x reference)
import functools

import jax
import jax.numpy as jnp
from jax import lax
from jax.experimental import pallas as pl
from jax.experimental.pallas import tpu as pltpu, tpu_sc as plsc

_C = 85
_CHROWS = 256
_CHW = _CHROWS * _C
_NS = (4096, 1024, 256)
_BATCH_OUT = 5376 * _C
_L1_OUT_OFF = 4096 * _C
_L2_OUT_OFF = 5120 * _C


def _decode_chunk(vin, vout, row0, wlog, stride):
    def body(iv, _):
        q = iv * 16 + lax.iota(jnp.int32, 16)
        d = ((q.astype(jnp.float32) + 0.5) * (1.0 / _C)).astype(jnp.int32)
        c = q - d * _C
        row = row0 + d
        gx = (row & ((1 << wlog) - 1)).astype(jnp.float32)
        gy = (row >> wlog).astype(jnp.float32)
        g = jnp.where(c == 0, gx, gy)
        v = vin[pl.ds(iv * 16, 16)]
        xy = (v + g) * stride
        wh = jnp.exp(v) * stride
        vout[pl.ds(iv * 16, 16)] = jnp.where(
            c < 2, xy, jnp.where(c < 4, wh, v)
        )
        return 0

    lax.fori_loop(0, _CHW // 16, body, 0, unroll=8)


def _sc_decode(f0, f1, f2, out, vin, vout):
    wid = lax.axis_index("s") * 2 + lax.axis_index("c")

    b0 = wid >> 1
    half = wid & 1

    def l0_chunk(k, _):
        in_off = b0 * (_NS[0] * _C) + half * (8 * _CHW) + k * _CHW
        out_off = b0 * _BATCH_OUT + half * (8 * _CHW) + k * _CHW
        pltpu.sync_copy(f0.at[pl.ds(in_off, _CHW)], vin)
        _decode_chunk(vin, vout, half * 2048 + k * _CHROWS, 6, 8.0)
        pltpu.sync_copy(vout, out.at[pl.ds(out_off, _CHW)])
        return 0

    lax.fori_loop(0, 8, l0_chunk, 0)

    @pl.when(wid < 16)
    def _():
        def l1_chunk(k, _):
            in_off = wid * (_NS[1] * _C) + k * _CHW
            out_off = wid * _BATCH_OUT + _L1_OUT_OFF + k * _CHW
            pltpu.sync_copy(f1.at[pl.ds(in_off, _CHW)], vin)
            _decode_chunk(vin, vout, k * _CHROWS, 5, 16.0)
            pltpu.sync_copy(vout, out.at[pl.ds(out_off, _CHW)])
            return 0

        lax.fori_loop(0, 4, l1_chunk, 0)

    @pl.when(wid >= 16)
    def _():
        b2 = wid - 16
        in_off = b2 * (_NS[2] * _C)
        out_off = b2 * _BATCH_OUT + _L2_OUT_OFF
        pltpu.sync_copy(f2.at[pl.ds(in_off, _CHW)], vin)
        _decode_chunk(vin, vout, 0, 4, 32.0)
        pltpu.sync_copy(vout, out.at[pl.ds(out_off, _CHW)])


@jax.jit
def kernel(feat0, feat1, feat2, targets):
    b = feat0.shape[0]
    f0 = feat0.reshape(b * _NS[0] * _C)
    f1 = feat1.reshape(b * _NS[1] * _C)
    f2 = feat2.reshape(b * _NS[2] * _C)
    run = functools.partial(
        pl.kernel,
        out_type=jax.ShapeDtypeStruct((b * _BATCH_OUT,), jnp.float32),
        mesh=plsc.VectorSubcoreMesh(core_axis_name="c", subcore_axis_name="s"),
        scratch_types=[
            pltpu.VMEM((_CHW,), jnp.float32),
            pltpu.VMEM((_CHW,), jnp.float32),
        ],
    )(_sc_decode)
    out = run(f0, f1, f2)
    return out.reshape(b, 5376, _C)


# speedup vs baseline: 6.1548x; 6.1548x over previous
import jax
import jax.numpy as jnp
from jax.experimental import pallas as pl

_NS = (4096, 1024, 256)
_NTOT = 5376
_C = 85
_BB = 4
_H0 = 2048


def _decode_level(v, stride, w, row0):
    n = v.shape[1]
    p = row0 + jax.lax.broadcasted_iota(jnp.int32, (1, n, 1), 1)
    gx = (p & (w - 1)).astype(jnp.float32)
    gy = (p // w).astype(jnp.float32)
    c = jax.lax.broadcasted_iota(jnp.int32, (1, n, _C), 2)
    g = jnp.where(c == 0, gx, gy)
    xy = (v + g) * stride
    wh = jnp.exp(v) * stride
    return jnp.where(c < 2, xy, jnp.where(c < 4, wh, v))


def _decode_kernel(f0a_ref, f0b_ref, f1_ref, f2_ref, out_ref):
    out_ref[:, pl.ds(0, _H0), :] = _decode_level(f0a_ref[:], 8.0, 64, 0)
    out_ref[:, pl.ds(_H0, _H0), :] = _decode_level(f0b_ref[:], 8.0, 64, _H0)
    out_ref[:, pl.ds(4096, 1024), :] = _decode_level(f1_ref[:], 16.0, 32, 0)
    out_ref[:, pl.ds(5120, 256), :] = _decode_level(f2_ref[:], 32.0, 16, 0)


@jax.jit
def kernel(feat0, feat1, feat2, targets):
    b = feat0.shape[0]
    f0 = feat0.reshape(b, _NS[0], _C)
    f1 = feat1.reshape(b, _NS[1], _C)
    f2 = feat2.reshape(b, _NS[2], _C)
    return pl.pallas_call(
        _decode_kernel,
        grid=(b // _BB,),
        in_specs=[
            pl.BlockSpec((_BB, _H0, _C), lambda i: (i, 0, 0)),
            pl.BlockSpec((_BB, _H0, _C), lambda i: (i, 1, 0)),
            pl.BlockSpec((_BB, _NS[1], _C), lambda i: (i, 0, 0)),
            pl.BlockSpec((_BB, _NS[2], _C), lambda i: (i, 0, 0)),
        ],
        out_specs=pl.BlockSpec((_BB, _NTOT, _C), lambda i: (i, 0, 0)),
        out_shape=jax.ShapeDtypeStruct((b, _NTOT, _C), jnp.float32),
    )(f0, f0, f1, f2)


# speedup vs baseline: 6.1989x; 1.0072x over previous
import jax
import jax.numpy as jnp
from jax.experimental import pallas as pl

_STRIDES = (8.0, 16.0, 32.0)
_WS = (64, 32, 16)
_NS = (4096, 1024, 256)
_OFFS = (0, 4096, 5120)
_NTOT = 5376
_C = 85
_BB = 4


def _decode_level(v, stride, w):
    n = v.shape[1]
    p = jax.lax.broadcasted_iota(jnp.int32, (1, n, 1), 1)
    gx = (p & (w - 1)).astype(jnp.float32)
    gy = (p // w).astype(jnp.float32)
    c = jax.lax.broadcasted_iota(jnp.int32, (1, n, _C), 2)
    g = jnp.where(c == 0, gx, gy)
    xy = (v + g) * stride
    wh = jnp.exp(v) * stride
    return jnp.where(c < 2, xy, jnp.where(c < 4, wh, v))


def _decode_kernel(f0_ref, f1_ref, f2_ref, out_ref):
    for ref, stride, w, off, n in zip(
        (f0_ref, f1_ref, f2_ref), _STRIDES, _WS, _OFFS, _NS
    ):
        out_ref[:, pl.ds(off, n), :] = _decode_level(ref[:], stride, w)


@jax.jit
def kernel(feat0, feat1, feat2, targets):
    b = feat0.shape[0]
    f0 = feat0.reshape(b, _NS[0], _C)
    f1 = feat1.reshape(b, _NS[1], _C)
    f2 = feat2.reshape(b, _NS[2], _C)
    return pl.pallas_call(
        _decode_kernel,
        grid=(b // _BB,),
        in_specs=[
            pl.BlockSpec((_BB, _NS[0], _C), lambda i: (i, 0, 0)),
            pl.BlockSpec((_BB, _NS[1], _C), lambda i: (i, 0, 0)),
            pl.BlockSpec((_BB, _NS[2], _C), lambda i: (i, 0, 0)),
        ],
        out_specs=pl.BlockSpec((_BB, _NTOT, _C), lambda i: (i, 0, 0)),
        out_shape=jax.ShapeDtypeStruct((b, _NTOT, _C), jnp.float32),
    )(f0, f1, f2)
